# baseline (device time: 19775 ns/iter reference)
import functools

import jax
import jax.numpy as jnp
import numpy as np
from jax import lax
from jax.experimental import pallas as pl
from jax.experimental.pallas import tpu as pltpu

N_DEV = 4
DH = 64


def _proj_allreduce_2phase(ctx, Wo):
    M, _ = ctx.shape
    H = Wo.shape[1] // 2

    def body(ctx_ref, wo_ref, out_ref, bufs, send_sems, recv_sems):
        my = lax.axis_index("i")
        pA = my ^ 1
        pB = 3 - my

        barrier_sem = pltpu.get_barrier_semaphore()
        for nbr in [pA, pB]:
            pl.semaphore_signal(
                barrier_sem, inc=1,
                device_id=(nbr,), device_id_type=pl.DeviceIdType.MESH,
            )

        bufs[0] = jnp.dot(
            ctx_ref[...], wo_ref[:, :H], preferred_element_type=jnp.float32
        )
        pl.semaphore_wait(barrier_sem, 2)

        l1 = pltpu.make_async_remote_copy(
            src_ref=bufs.at[0], dst_ref=bufs.at[2],
            send_sem=send_sems.at[0], recv_sem=recv_sems.at[0],
            device_id=(pA,), device_id_type=pl.DeviceIdType.MESH,
        )
        l1.start()
        bufs[1] = jnp.dot(
            ctx_ref[...], wo_ref[:, H:], preferred_element_type=jnp.float32
        )
        r1 = pltpu.make_async_remote_copy(
            src_ref=bufs.at[1], dst_ref=bufs.at[3],
            send_sem=send_sems.at[1], recv_sem=recv_sems.at[1],
            device_id=(pB,), device_id_type=pl.DeviceIdType.MESH,
        )
        r1.start()

        l1.wait()
        bufs[4] = bufs[0] + bufs[2]
        l2 = pltpu.make_async_remote_copy(
            src_ref=bufs.at[4], dst_ref=bufs.at[6],
            send_sem=send_sems.at[2], recv_sem=recv_sems.at[2],
            device_id=(pB,), device_id_type=pl.DeviceIdType.MESH,
        )
        l2.start()
        r1.wait()
        bufs[5] = bufs[1] + bufs[3]
        r2 = pltpu.make_async_remote_copy(
            src_ref=bufs.at[5], dst_ref=bufs.at[7],
            send_sem=send_sems.at[3], recv_sem=recv_sems.at[3],
            device_id=(pA,), device_id_type=pl.DeviceIdType.MESH,
        )
        r2.start()

        l2.wait()
        out_ref[:, :H] = bufs[4] + bufs[6]
        r2.wait()
        out_ref[:, H:] = bufs[5] + bufs[7]

    return pl.pallas_call(
        body,
        out_shape=jax.ShapeDtypeStruct((M, 2 * H), jnp.float32),
        in_specs=[
            pl.BlockSpec(memory_space=pltpu.VMEM),
            pl.BlockSpec(memory_space=pltpu.VMEM),
        ],
        out_specs=pl.BlockSpec(memory_space=pltpu.VMEM),
        scratch_shapes=[
            pltpu.VMEM((8, M, H), jnp.float32),
            pltpu.SemaphoreType.DMA((4,)),
            pltpu.SemaphoreType.DMA((4,)),
        ],
        compiler_params=pltpu.CompilerParams(collective_id=0),
    )(ctx, Wo)


def kernel(x, Wq, Wk, Wv, Wo):
    B, Sq, D = x.shape
    Hl = Wq.shape[1] // DH

    xf = x.reshape(B * Sq, D)
    q = (xf @ Wq).reshape(B, Sq, Hl, DH)
    k = (xf @ Wk).reshape(B, Sq, Hl, DH)
    v = (xf @ Wv).reshape(B, Sq, Hl, DH)

    inv = 1.0 / (10000.0 ** (np.arange(0, DH, 2) / DH))
    pos = np.arange(Sq)[:, None] * inv[None, :]
    cos = jnp.asarray(np.repeat(np.cos(pos), 2, axis=-1).astype(np.float32))
    sin = jnp.asarray(np.repeat(np.sin(pos), 2, axis=-1).astype(np.float32))
    cos = cos[None, :, None, :]
    sin = sin[None, :, None, :]

    def rot(t):
        t2 = t.reshape(B, Sq, Hl, DH // 2, 2)
        t_r = jnp.stack([-t2[..., 1], t2[..., 0]], axis=-1).reshape(B, Sq, Hl, DH)
        return t * cos + t_r * sin

    Q = rot(q)
    K = rot(k)
    s = jnp.einsum("bihd,bjhd->bhij", Q, K) * 0.125
    s_max = jnp.max(s, axis=-1, keepdims=True)
    w = jnp.exp(s - s_max)
    w = w / jnp.sum(w, axis=-1, keepdims=True)
    ctx = jnp.einsum("bhij,bjhd->bihd", w, v).reshape(B * Sq, Hl * DH)

    out = _proj_allreduce_2phase(ctx, Wo)
    return out.reshape(B, Sq, D)
